# Initial kernel scaffold; baseline (speedup 1.0000x reference)
import jax
import jax.numpy as jnp
from jax.experimental import pallas as pl


def kernel(ndata, edge_index, polar):
    raise NotImplementedError("write your pallas kernel here")


# speedup vs baseline: 7.3944x; 7.3944x over previous
import functools

import jax
import jax.numpy as jnp
from jax import lax
from jax.experimental import pallas as pl
from jax.experimental.pallas import tpu as pltpu
from jax.experimental.pallas import tpu_sc as plsc

N = 100000
E = 3200000
D = 16

NC = 2
NS = 16
G = 128
NG = E // G
CHUNK = 8
NCH = NG // CHUNK

NP = 100032

ZSLICE = 6256
ZTAIL = N - 15 * ZSLICE


def _sc_segment_sum(dst2d, polar, zsum):
    mesh = plsc.VectorSubcoreMesh(core_axis_name="c", subcore_axis_name="s")

    @functools.partial(
        pl.kernel,
        out_type=[
            jax.ShapeDtypeStruct((NC, NP, D), jnp.float32),
            jax.ShapeDtypeStruct((N,), jnp.float32),
            jax.ShapeDtypeStruct((N,), jnp.float32),
        ],
        mesh=mesh,
        compiler_params=pltpu.CompilerParams(use_tc_tiling_on_sc=False),
        scratch_types=[
            pltpu.VMEM((CHUNK, G), jnp.int32),
            pltpu.VMEM((CHUNK * G, D), jnp.float32),
            pltpu.VMEM((G,), jnp.float32),
            pltpu.VMEM((ZSLICE,), jnp.float32),
            pltpu.VMEM_SHARED((N, D), jnp.float32),
            pltpu.VMEM_SHARED((N,), jnp.float32),
        ],
    )
    def k(dst_hbm, polar_hbm, zsum_hbm, sums_out, cnt0_out, cnt1_out,
          idx_v, rows_v, ones_v, stage_v, acc_s, acc_c):
        c = lax.axis_index("c")
        s = lax.axis_index("s")

        one16 = jnp.ones((16,), jnp.float32)
        for j in range(G // 16):
            ones_v[pl.ds(j * 16, 16)] = one16

        zero16 = jnp.zeros((16,), jnp.float32)

        @pl.loop(0, ZSLICE // 16)
        def _(j):
            stage_v[pl.ds(j * 16, 16)] = zero16

        off = s * ZSLICE

        @pl.when(s < NS - 1)
        def _():
            pltpu.sync_copy(zsum_hbm.at[pl.ds(off, ZSLICE)],
                            acc_s.at[pl.ds(off, ZSLICE)])
            pltpu.sync_copy(stage_v, acc_c.at[pl.ds(off, ZSLICE)])

        @pl.when(s == NS - 1)
        def _():
            pltpu.sync_copy(zsum_hbm.at[pl.ds(off, ZTAIL)],
                            acc_s.at[pl.ds(off, ZTAIL)])
            pltpu.sync_copy(stage_v.at[pl.ds(0, ZTAIL)],
                            acc_c.at[pl.ds(off, ZTAIL)])

        plsc.subcore_barrier()

        half = NCH // 2
        core_base = jnp.where(c == 0, 0, half)
        core_n = jnp.where(c == 0, half, NCH - half)
        ch0 = core_base + (s * core_n) // NS
        ch1 = core_base + ((s + 1) * core_n) // NS

        def chunk_body(t, carry):
            base = t * CHUNK
            pltpu.sync_copy(dst_hbm.at[pl.ds(base, CHUNK)], idx_v)
            pltpu.sync_copy(polar_hbm.at[pl.ds(base * G, CHUNK * G)], rows_v)
            for j in range(CHUNK):
                pltpu.sync_copy(rows_v.at[pl.ds(j * G, G)],
                                acc_s.at[idx_v.at[j]], add=True)
                pltpu.sync_copy(ones_v, acc_c.at[idx_v.at[j]], add=True)
            return carry

        lax.fori_loop(ch0, ch1, chunk_body, 0)

        plsc.subcore_barrier()

        sl = pl.ds(off, ZSLICE)
        tl = pl.ds(off, ZTAIL)

        @pl.when(s < NS - 1)
        def _():
            pltpu.sync_copy(acc_s.at[sl], sums_out.at[c, sl])
            pltpu.sync_copy(acc_c.at[sl], stage_v)

            @pl.when(c == 0)
            def _():
                pltpu.sync_copy(stage_v, cnt0_out.at[sl])

            @pl.when(c == 1)
            def _():
                pltpu.sync_copy(stage_v, cnt1_out.at[sl])

        @pl.when(s == NS - 1)
        def _():
            pltpu.sync_copy(acc_s.at[tl], sums_out.at[c, tl])
            pltpu.sync_copy(acc_c.at[tl], stage_v.at[pl.ds(0, ZTAIL)])

            @pl.when(c == 0)
            def _():
                pltpu.sync_copy(stage_v.at[pl.ds(0, ZTAIL)], cnt0_out.at[tl])

            @pl.when(c == 1)
            def _():
                pltpu.sync_copy(stage_v.at[pl.ds(0, ZTAIL)], cnt1_out.at[tl])

    return k(dst2d, polar, zsum)


BN = 4000


def _combine_body(sums_ref, cnt0_ref, cnt1_ref, nd_ref, out_ref):
    ssum = sums_ref[0] + sums_ref[1]
    cnt = (cnt0_ref[0, 0, :] + cnt1_ref[0, 0, :])[:, None]
    mean = ssum / jnp.maximum(cnt, 1.0)
    out_ref[...] = jnp.where(cnt > 0.0, mean, nd_ref[...])


def _combine(sums, cnt0, cnt1, ndata):
    return pl.pallas_call(
        _combine_body,
        grid=(N // BN,),
        in_specs=[
            pl.BlockSpec((NC, BN, D), lambda i: (0, i, 0)),
            pl.BlockSpec((1, 1, BN), lambda i: (i, 0, 0)),
            pl.BlockSpec((1, 1, BN), lambda i: (i, 0, 0)),
            pl.BlockSpec((BN, D), lambda i: (i, 0)),
        ],
        out_specs=pl.BlockSpec((BN, D), lambda i: (i, 0)),
        out_shape=jax.ShapeDtypeStruct((N, D), jnp.float32),
    )(sums, cnt0.reshape(N // BN, 1, BN), cnt1.reshape(N // BN, 1, BN), ndata)


def kernel(ndata, edge_index, polar):
    dst2d = edge_index[1].reshape(NG, G)
    zsum = jnp.zeros((N, D), jnp.float32)
    sums, cnt0, cnt1 = _sc_segment_sum(dst2d, polar, zsum)
    return _combine(sums, cnt0, cnt1, ndata)


# speedup vs baseline: 8.0678x; 1.0911x over previous
import functools

import jax
import jax.numpy as jnp
from jax import lax
from jax.experimental import pallas as pl
from jax.experimental.pallas import tpu as pltpu
from jax.experimental.pallas import tpu_sc as plsc

N = 100000
E = 3200000
D = 16

NC = 2
NS = 16
G = 128
NG = E // G
CHUNK = 8
NCH = NG // CHUNK

NP = 100032

ZSLICE = 6256
ZTAIL = N - 15 * ZSLICE


def _sc_segment_sum(dst1d, polar, zsum):
    mesh = plsc.VectorSubcoreMesh(core_axis_name="c", subcore_axis_name="s")

    @functools.partial(
        pl.kernel,
        out_type=[
            jax.ShapeDtypeStruct((NC, NP, D), jnp.float32),
            jax.ShapeDtypeStruct((N,), jnp.float32),
            jax.ShapeDtypeStruct((N,), jnp.float32),
        ],
        mesh=mesh,
        compiler_params=pltpu.CompilerParams(use_tc_tiling_on_sc=False),
        scratch_types=[
            pltpu.VMEM((CHUNK * G,), jnp.int32),
            pltpu.VMEM((CHUNK * G, D), jnp.float32),
            pltpu.VMEM((CHUNK * G,), jnp.float32),
            pltpu.VMEM_SHARED((N, D), jnp.float32),
            pltpu.VMEM_SHARED((N,), jnp.float32),
            pltpu.SemaphoreType.DMA,
            pltpu.SemaphoreType.DMA,
        ],
    )
    def k(dst_hbm, polar_hbm, zsum_hbm, zcnt_hbm, sums_out, cnt0_out,
          cnt1_out, idx_v, rows_v, ones_v, acc_s, acc_c, ld_sem, st_sem):
        c = lax.axis_index("c")
        s = lax.axis_index("s")

        one16 = jnp.ones((16,), jnp.float32)

        @pl.loop(0, (CHUNK * G) // 16)
        def _(j):
            ones_v[pl.ds(j * 16, 16)] = one16

        off = s * ZSLICE

        @pl.when(s < NS - 1)
        def _():
            pltpu.sync_copy(zsum_hbm.at[pl.ds(off, ZSLICE)],
                            acc_s.at[pl.ds(off, ZSLICE)])
            pltpu.sync_copy(zcnt_hbm.at[pl.ds(off, ZSLICE)],
                            acc_c.at[pl.ds(off, ZSLICE)])

        @pl.when(s == NS - 1)
        def _():
            pltpu.sync_copy(zsum_hbm.at[pl.ds(off, ZTAIL)],
                            acc_s.at[pl.ds(off, ZTAIL)])
            pltpu.sync_copy(zcnt_hbm.at[pl.ds(off, ZTAIL)],
                            acc_c.at[pl.ds(off, ZTAIL)])

        plsc.subcore_barrier()

        half = NCH // 2
        core_base = jnp.where(c == 0, 0, half)
        core_n = jnp.where(c == 0, half, NCH - half)
        ch0 = core_base + (s * core_n) // NS
        ch1 = core_base + ((s + 1) * core_n) // NS

        def chunk_body(t, carry):
            base = t * CHUNK * G
            a = pltpu.async_copy(dst_hbm.at[pl.ds(base, CHUNK * G)],
                                 idx_v, ld_sem)
            b = pltpu.async_copy(polar_hbm.at[pl.ds(base, CHUNK * G)],
                                 rows_v, ld_sem)
            a.wait()
            b.wait()
            p = pltpu.async_copy(rows_v, acc_s.at[idx_v], st_sem, add=True)
            q = pltpu.async_copy(ones_v, acc_c.at[idx_v], st_sem, add=True)
            p.wait()
            q.wait()
            return carry

        lax.fori_loop(ch0, ch1, chunk_body, 0)

        plsc.subcore_barrier()

        sl = pl.ds(off, ZSLICE)
        tl = pl.ds(off, ZTAIL)

        @pl.when(s < NS - 1)
        def _():
            pltpu.sync_copy(acc_s.at[sl], sums_out.at[c, sl])

            @pl.when(c == 0)
            def _():
                pltpu.sync_copy(acc_c.at[sl], cnt0_out.at[sl])

            @pl.when(c == 1)
            def _():
                pltpu.sync_copy(acc_c.at[sl], cnt1_out.at[sl])

        @pl.when(s == NS - 1)
        def _():
            pltpu.sync_copy(acc_s.at[tl], sums_out.at[c, tl])

            @pl.when(c == 0)
            def _():
                pltpu.sync_copy(acc_c.at[tl], cnt0_out.at[tl])

            @pl.when(c == 1)
            def _():
                pltpu.sync_copy(acc_c.at[tl], cnt1_out.at[tl])

    zcnt = jnp.zeros((N,), jnp.float32)
    return k(dst1d, polar, zsum, zcnt)


BN = 4000


def _combine_body(sums_ref, cnt0_ref, cnt1_ref, nd_ref, out_ref):
    ssum = sums_ref[0] + sums_ref[1]
    cnt = (cnt0_ref[0, 0, :] + cnt1_ref[0, 0, :])[:, None]
    mean = ssum / jnp.maximum(cnt, 1.0)
    out_ref[...] = jnp.where(cnt > 0.0, mean, nd_ref[...])


def _combine(sums, cnt0, cnt1, ndata):
    return pl.pallas_call(
        _combine_body,
        grid=(N // BN,),
        in_specs=[
            pl.BlockSpec((NC, BN, D), lambda i: (0, i, 0)),
            pl.BlockSpec((1, 1, BN), lambda i: (i, 0, 0)),
            pl.BlockSpec((1, 1, BN), lambda i: (i, 0, 0)),
            pl.BlockSpec((BN, D), lambda i: (i, 0)),
        ],
        out_specs=pl.BlockSpec((BN, D), lambda i: (i, 0)),
        out_shape=jax.ShapeDtypeStruct((N, D), jnp.float32),
    )(sums, cnt0.reshape(N // BN, 1, BN), cnt1.reshape(N // BN, 1, BN), ndata)


def kernel(ndata, edge_index, polar):
    dst1d = edge_index[1]
    zsum = jnp.zeros((N, D), jnp.float32)
    sums, cnt0, cnt1 = _sc_segment_sum(dst1d, polar, zsum)
    return _combine(sums, cnt0, cnt1, ndata)
